# Initial kernel scaffold; baseline (speedup 1.0000x reference)
import functools

import jax
import jax.numpy as jnp
from jax import lax
from jax.experimental import pallas as pl
from jax.experimental.pallas import tpu as pltpu
from jax.experimental.pallas import tpu_sc as plsc

N_NODES = 10000
D = 128
N_EDGES = 320000

NC = 2
NS = 16
CHUNK = 128
CPW = 80
E_PAD = NC * NS * CPW * CHUNK
N_PAD = 10240
ROWS_PS = N_PAD // NS

_MESH = dict(core_axis_name="c", subcore_axis_name="s", num_cores=NC,
             num_subcores=NS)


def _deg_body(dst_hbm, out_hbm, dst_v, buf, acc_sh):
    ci = lax.axis_index("c")
    si = lax.axis_index("s")
    pltpu.sync_copy(dst_hbm.at[ci, si], dst_v)
    @pl.loop(0, ROWS_PS)
    def _(r):
        buf[r, :] = jnp.zeros((16,), jnp.float32)
    pltpu.sync_copy(buf, acc_sh.at[pl.ds(si * ROWS_PS, ROWS_PS)])
    @pl.loop(0, CHUNK)
    def _(r):
        buf[r, :] = jnp.ones((16,), jnp.float32)
    plsc.subcore_barrier()
    @pl.loop(0, CPW)
    def _(c):
        pltpu.sync_copy(buf.at[pl.ds(0, CHUNK)], acc_sh.at[dst_v.at[c]],
                        add=True)
    plsc.subcore_barrier()
    pltpu.sync_copy(acc_sh.at[pl.ds(si * ROWS_PS, ROWS_PS)],
                    out_hbm.at[ci].at[pl.ds(si * ROWS_PS, ROWS_PS)])


def _agg_body(g_hbm, src_hbm, dst_hbm, out_hbm, src_v, dst_v, rows, acc_sh):
    ci = lax.axis_index("c")
    si = lax.axis_index("s")
    pltpu.sync_copy(src_hbm.at[ci, si], src_v)
    pltpu.sync_copy(dst_hbm.at[ci, si], dst_v)
    @pl.loop(0, CHUNK)
    def _(r):
        @pl.loop(0, D, step=16)
        def _(j):
            rows[r, pl.ds(j, 16)] = jnp.zeros((16,), jnp.float32)
    for z in range(ROWS_PS // CHUNK):
        pltpu.sync_copy(rows, acc_sh.at[pl.ds(si * ROWS_PS + z * CHUNK, CHUNK)])
    plsc.subcore_barrier()
    @pl.loop(0, CPW)
    def _(c):
        pltpu.sync_copy(g_hbm.at[src_v.at[c]], rows)
        pltpu.sync_copy(rows, acc_sh.at[dst_v.at[c]], add=True)
    plsc.subcore_barrier()
    pltpu.sync_copy(acc_sh.at[pl.ds(si * ROWS_PS, ROWS_PS)],
                    out_hbm.at[ci].at[pl.ds(si * ROWS_PS, ROWS_PS)])


def _sc_deg(dst_a):
    return pl.kernel(
        _deg_body,
        out_type=jax.ShapeDtypeStruct((NC, N_PAD, 16), jnp.float32),
        mesh=plsc.VectorSubcoreMesh(**_MESH),
        scratch_types=[
            pltpu.VMEM((CPW, CHUNK), jnp.int32),
            pltpu.VMEM((ROWS_PS, 16), jnp.float32),
            pltpu.VMEM_SHARED((N_PAD, 16), jnp.float32),
        ],
    )(dst_a)


def _sc_agg(g, src_a, dst_a):
    return pl.kernel(
        _agg_body,
        out_type=jax.ShapeDtypeStruct((NC, N_PAD, D), jnp.float32),
        mesh=plsc.VectorSubcoreMesh(**_MESH),
        scratch_types=[
            pltpu.VMEM((CPW, CHUNK), jnp.int32),
            pltpu.VMEM((CPW, CHUNK), jnp.int32),
            pltpu.VMEM((CHUNK, D), jnp.float32),
            pltpu.VMEM_SHARED((N_PAD, D), jnp.float32),
        ],
    )(g, src_a, dst_a)


def _mm_body(x_ref, w_ref, o_ref):
    o_ref[...] = jnp.dot(x_ref[...], w_ref[...],
                         preferred_element_type=jnp.float32)


def _disg_body(h_ref, dp_ref, g_ref, dis_ref):
    deg = dp_ref[0, :, 0:1] + dp_ref[1, :, 0:1] + 1.0
    dis = lax.rsqrt(deg)
    dis_ref[...] = dis
    g_ref[...] = h_ref[...] * dis


def _mid_body(p_ref, g1_ref, dis_ref, b1_ref, w2_ref, g2_ref):
    agg = p_ref[0] + p_ref[1] + g1_ref[...]
    h = jnp.maximum(agg * dis_ref[...] + b1_ref[...], 0.0)
    h2 = jnp.dot(h, w2_ref[...], preferred_element_type=jnp.float32)
    g2_ref[...] = h2 * dis_ref[...]


def _fin_body(p_ref, g2_ref, dis_ref, b2_ref, o_ref):
    o_ref[...] = ((p_ref[0] + p_ref[1] + g2_ref[...]) * dis_ref[...]
                  + b2_ref[...])


def _tc(body, *out_shapes):
    return pl.pallas_call(
        body,
        out_shape=[jax.ShapeDtypeStruct(s, jnp.float32) for s in out_shapes],
    )


@jax.jit
def kernel(x, edge_index, W1, b1, W2, b2):
    ei = edge_index.astype(jnp.int32)
    pad = jnp.full((E_PAD - N_EDGES,), N_NODES, jnp.int32)
    src_a = jnp.concatenate([ei[0], pad]).reshape(NC, NS, CPW, CHUNK)
    dst_a = jnp.concatenate([ei[1], pad]).reshape(NC, NS, CPW, CHUNK)
    x_pad = jnp.pad(x, ((0, N_PAD - N_NODES), (0, 0)))
    b1r = b1.reshape(1, D)
    b2r = b2.reshape(1, D)

    degp = _sc_deg(dst_a)
    (h1,) = _tc(_mm_body, (N_PAD, D))(x_pad, W1)
    g1, dis = _tc(_disg_body, (N_PAD, D), (N_PAD, 1))(h1, degp)
    p1 = _sc_agg(g1, src_a, dst_a)
    (g2,) = _tc(_mid_body, (N_PAD, D))(p1, g1, dis, b1r, W2)
    p2 = _sc_agg(g2, src_a, dst_a)
    (out,) = _tc(_fin_body, (N_PAD, D))(p2, g2, dis, b2r)
    return out[:N_NODES]


# speedup vs baseline: 8.8278x; 8.8278x over previous
import functools

import jax
import jax.numpy as jnp
from jax import lax
from jax.experimental import pallas as pl
from jax.experimental.pallas import tpu as pltpu
from jax.experimental.pallas import tpu_sc as plsc

N_NODES = 10000
D = 128
N_EDGES = 320000

NC = 2
NS = 16
CHUNK = 128
CPW = 80
E_PAD = NC * NS * CPW * CHUNK
N_PAD = 10240
ROWS_PS = N_PAD // NS

_MESH = dict(core_axis_name="c", subcore_axis_name="s", num_cores=NC,
             num_subcores=NS)


def _deg_body(dst_hbm, out_hbm, dst_v, ones_v, zbuf, acc_sh):
    ci = lax.axis_index("c")
    si = lax.axis_index("s")
    pltpu.sync_copy(dst_hbm.at[ci, si], dst_v)
    @pl.loop(0, CHUNK)
    def _(r):
        @pl.loop(0, D, step=16)
        def _(j):
            zbuf[r, pl.ds(j, 16)] = jnp.zeros((16,), jnp.float32)
            ones_v[r, pl.ds(j, 16)] = jnp.ones((16,), jnp.float32)
    for z in range(ROWS_PS // CHUNK):
        pltpu.sync_copy(zbuf, acc_sh.at[pl.ds(si * ROWS_PS + z * CHUNK, CHUNK)])
    plsc.subcore_barrier()
    @pl.loop(0, CPW)
    def _(c):
        pltpu.sync_copy(ones_v, acc_sh.at[dst_v.at[c]], add=True)
    plsc.subcore_barrier()
    pltpu.sync_copy(acc_sh.at[pl.ds(si * ROWS_PS, ROWS_PS)],
                    out_hbm.at[ci].at[pl.ds(si * ROWS_PS, ROWS_PS)])


def _agg_body(g_hbm, src_hbm, dst_hbm, out_hbm, src_v, dst_v, rows, acc_sh):
    ci = lax.axis_index("c")
    si = lax.axis_index("s")
    pltpu.sync_copy(src_hbm.at[ci, si], src_v)
    pltpu.sync_copy(dst_hbm.at[ci, si], dst_v)
    @pl.loop(0, CHUNK)
    def _(r):
        @pl.loop(0, D, step=16)
        def _(j):
            rows[r, pl.ds(j, 16)] = jnp.zeros((16,), jnp.float32)
    for z in range(ROWS_PS // CHUNK):
        pltpu.sync_copy(rows, acc_sh.at[pl.ds(si * ROWS_PS + z * CHUNK, CHUNK)])
    plsc.subcore_barrier()
    @pl.loop(0, CPW)
    def _(c):
        pltpu.sync_copy(g_hbm.at[src_v.at[c]], rows)
        pltpu.sync_copy(rows, acc_sh.at[dst_v.at[c]], add=True)
    plsc.subcore_barrier()
    pltpu.sync_copy(acc_sh.at[pl.ds(si * ROWS_PS, ROWS_PS)],
                    out_hbm.at[ci].at[pl.ds(si * ROWS_PS, ROWS_PS)])


def _sc_deg(dst_a):
    return pl.kernel(
        _deg_body,
        out_type=jax.ShapeDtypeStruct((NC, N_PAD, D), jnp.float32),
        mesh=plsc.VectorSubcoreMesh(**_MESH),
        scratch_types=[
            pltpu.VMEM((CPW, CHUNK), jnp.int32),
            pltpu.VMEM((CHUNK, D), jnp.float32),
            pltpu.VMEM((CHUNK, D), jnp.float32),
            pltpu.VMEM_SHARED((N_PAD, D), jnp.float32),
        ],
    )(dst_a)


def _sc_agg(g, src_a, dst_a):
    return pl.kernel(
        _agg_body,
        out_type=jax.ShapeDtypeStruct((NC, N_PAD, D), jnp.float32),
        mesh=plsc.VectorSubcoreMesh(**_MESH),
        scratch_types=[
            pltpu.VMEM((CPW, CHUNK), jnp.int32),
            pltpu.VMEM((CPW, CHUNK), jnp.int32),
            pltpu.VMEM((CHUNK, D), jnp.float32),
            pltpu.VMEM_SHARED((N_PAD, D), jnp.float32),
        ],
    )(g, src_a, dst_a)


def _mm_body(x_ref, w_ref, o_ref):
    o_ref[...] = jnp.dot(x_ref[...], w_ref[...],
                         preferred_element_type=jnp.float32)


def _disg_body(h_ref, dp_ref, g_ref, dis_ref):
    deg = dp_ref[0, :, 0:1] + dp_ref[1, :, 0:1] + 1.0
    dis = lax.rsqrt(deg)
    dis_ref[...] = dis
    g_ref[...] = h_ref[...] * dis


def _mid_body(p_ref, g1_ref, dis_ref, b1_ref, w2_ref, g2_ref):
    agg = p_ref[0] + p_ref[1] + g1_ref[...]
    h = jnp.maximum(agg * dis_ref[...] + b1_ref[...], 0.0)
    h2 = jnp.dot(h, w2_ref[...], preferred_element_type=jnp.float32)
    g2_ref[...] = h2 * dis_ref[...]


def _fin_body(p_ref, g2_ref, dis_ref, b2_ref, o_ref):
    o_ref[...] = ((p_ref[0] + p_ref[1] + g2_ref[...]) * dis_ref[...]
                  + b2_ref[...])


def _tc(body, *out_shapes):
    return pl.pallas_call(
        body,
        out_shape=[jax.ShapeDtypeStruct(s, jnp.float32) for s in out_shapes],
    )


@jax.jit
def kernel(x, edge_index, W1, b1, W2, b2):
    ei = edge_index.astype(jnp.int32)
    pad = jnp.full((E_PAD - N_EDGES,), N_NODES, jnp.int32)
    src_a = jnp.concatenate([ei[0], pad]).reshape(NC, NS, CPW, CHUNK)
    dst_a = jnp.concatenate([ei[1], pad]).reshape(NC, NS, CPW, CHUNK)
    x_pad = jnp.pad(x, ((0, N_PAD - N_NODES), (0, 0)))
    b1r = b1.reshape(1, D)
    b2r = b2.reshape(1, D)

    degp = _sc_deg(dst_a)
    (h1,) = _tc(_mm_body, (N_PAD, D))(x_pad, W1)
    g1, dis = _tc(_disg_body, (N_PAD, D), (N_PAD, 1))(h1, degp)
    p1 = _sc_agg(g1, src_a, dst_a)
    (g2,) = _tc(_mid_body, (N_PAD, D))(p1, g1, dis, b1r, W2)
    p2 = _sc_agg(g2, src_a, dst_a)
    (out,) = _tc(_fin_body, (N_PAD, D))(p2, g2, dis, b2r)
    return out[:N_NODES]


# speedup vs baseline: 9.3103x; 1.0547x over previous
import functools

import jax
import jax.numpy as jnp
from jax import lax
from jax.experimental import pallas as pl
from jax.experimental.pallas import tpu as pltpu
from jax.experimental.pallas import tpu_sc as plsc

N_NODES = 10000
D = 128
N_EDGES = 320000

NC = 2
NS = 16
CHUNK = 128
CPW = 80
HALF = CPW // 2
E_PAD = NC * NS * CPW * CHUNK
N_PAD = 10240
ROWS_PS = N_PAD // NS

_MESH = dict(core_axis_name="c", subcore_axis_name="s", num_cores=NC,
             num_subcores=NS)


def _deg_body(dst_hbm, out_hbm, dst_v, ones_v, zbuf, acc_sh):
    ci = lax.axis_index("c")
    si = lax.axis_index("s")
    pltpu.sync_copy(dst_hbm.at[ci, si], dst_v)
    @pl.loop(0, CHUNK)
    def _(r):
        @pl.loop(0, D, step=16)
        def _(j):
            zbuf[r, pl.ds(j, 16)] = jnp.zeros((16,), jnp.float32)
            ones_v[r, pl.ds(j, 16)] = jnp.ones((16,), jnp.float32)
    for z in range(ROWS_PS // CHUNK):
        pltpu.sync_copy(zbuf, acc_sh.at[pl.ds(si * ROWS_PS + z * CHUNK, CHUNK)])
    plsc.subcore_barrier()
    @pl.loop(0, CPW)
    def _(c):
        pltpu.sync_copy(ones_v, acc_sh.at[dst_v.at[c]], add=True)
    plsc.subcore_barrier()
    pltpu.sync_copy(acc_sh.at[pl.ds(si * ROWS_PS, ROWS_PS)],
                    out_hbm.at[ci].at[pl.ds(si * ROWS_PS, ROWS_PS)])


NBUF = 2


def _agg_body(g_hbm, ei_hbm, out_hbm, idx_v, rows, acc_sh, *sems):
    gsem = sems[:NBUF]
    ssem = sems[NBUF:]
    ci = lax.axis_index("c")
    si = lax.axis_index("s")
    @pl.loop(0, CHUNK)
    def _(r):
        @pl.loop(0, D, step=16)
        def _(j):
            rows[0, r, pl.ds(j, 16)] = jnp.zeros((16,), jnp.float32)
    for z in range(ROWS_PS // CHUNK):
        pltpu.sync_copy(rows.at[0],
                        acc_sh.at[pl.ds(si * ROWS_PS + z * CHUNK, CHUNK)])
    plsc.subcore_barrier()
    for h in range(CPW // HALF):
        pltpu.sync_copy(ei_hbm.at[ci, si, pl.ds(h * HALF, HALF)], idx_v)
        for b in range(NBUF):
            pltpu.async_copy(g_hbm.at[idx_v.at[b, 0]], rows.at[b], gsem[b])

        @pl.loop(0, HALF, step=NBUF)
        def _(c0):
            for b in range(NBUF):
                pltpu.make_async_copy(g_hbm.at[idx_v.at[c0 + b, 0]],
                                      rows.at[b], gsem[b]).wait()
                pltpu.async_copy(rows.at[b], acc_sh.at[idx_v.at[c0 + b, 1]],
                                 ssem[b], add=True)
            for b in range(NBUF):
                c = c0 + b + NBUF
                @pl.when(c < HALF)
                def _():
                    pltpu.make_async_copy(rows.at[b],
                                          acc_sh.at[idx_v.at[c0 + b, 1]],
                                          ssem[b]).wait()
                    pltpu.async_copy(g_hbm.at[idx_v.at[c, 0]], rows.at[b],
                                     gsem[b])

        for b in range(NBUF):
            pltpu.make_async_copy(rows.at[b],
                                  acc_sh.at[idx_v.at[HALF - NBUF + b, 1]],
                                  ssem[b]).wait()
    plsc.subcore_barrier()
    pltpu.sync_copy(acc_sh.at[pl.ds(si * ROWS_PS, ROWS_PS)],
                    out_hbm.at[ci].at[pl.ds(si * ROWS_PS, ROWS_PS)])


def _sc_deg(dst_a):
    return pl.kernel(
        _deg_body,
        out_type=jax.ShapeDtypeStruct((NC, N_PAD, D), jnp.float32),
        mesh=plsc.VectorSubcoreMesh(**_MESH),
        scratch_types=[
            pltpu.VMEM((CPW, CHUNK), jnp.int32),
            pltpu.VMEM((CHUNK, D), jnp.float32),
            pltpu.VMEM((CHUNK, D), jnp.float32),
            pltpu.VMEM_SHARED((N_PAD, D), jnp.float32),
        ],
    )(dst_a)


def _sc_agg(g, ei_a):
    return pl.kernel(
        _agg_body,
        out_type=jax.ShapeDtypeStruct((NC, N_PAD, D), jnp.float32),
        mesh=plsc.VectorSubcoreMesh(**_MESH),
        scratch_types=[
            pltpu.VMEM((HALF, 2, CHUNK), jnp.int32),
            pltpu.VMEM((NBUF, CHUNK, D), jnp.float32),
            pltpu.VMEM_SHARED((N_PAD, D), jnp.float32),
        ] + [pltpu.SemaphoreType.DMA] * (2 * NBUF),
    )(g, ei_a)


def _mm_body(x_ref, w_ref, o_ref):
    o_ref[...] = jnp.dot(x_ref[...], w_ref[...],
                         preferred_element_type=jnp.float32)


def _disg_body(h_ref, dp_ref, g_ref, dis_ref):
    deg = dp_ref[0, :, 0:1] + dp_ref[1, :, 0:1] + 1.0
    dis = lax.rsqrt(deg)
    dis_ref[...] = dis
    g_ref[...] = h_ref[...] * dis


def _mid_body(p_ref, g1_ref, dis_ref, b1_ref, w2_ref, g2_ref):
    agg = p_ref[0] + p_ref[1] + g1_ref[...]
    h = jnp.maximum(agg * dis_ref[...] + b1_ref[...], 0.0)
    h2 = jnp.dot(h, w2_ref[...], preferred_element_type=jnp.float32)
    g2_ref[...] = h2 * dis_ref[...]


def _fin_body(p_ref, g2_ref, dis_ref, b2_ref, o_ref):
    o_ref[...] = ((p_ref[0] + p_ref[1] + g2_ref[...]) * dis_ref[...]
                  + b2_ref[...])


def _tc(body, *out_shapes):
    return pl.pallas_call(
        body,
        out_shape=[jax.ShapeDtypeStruct(s, jnp.float32) for s in out_shapes],
    )


@jax.jit
def kernel(x, edge_index, W1, b1, W2, b2):
    ei = edge_index.astype(jnp.int32)
    pad = jnp.full((2, E_PAD - N_EDGES), N_NODES, jnp.int32)
    ei_a = (jnp.concatenate([ei, pad], axis=1)
            .reshape(2, NC, NS, CPW, CHUNK).transpose(1, 2, 3, 0, 4))
    x_pad = jnp.pad(x, ((0, N_PAD - N_NODES), (0, 0)))
    b1r = b1.reshape(1, D)
    b2r = b2.reshape(1, D)

    degp = _sc_deg(ei_a[:, :, :, 1])
    (h1,) = _tc(_mm_body, (N_PAD, D))(x_pad, W1)
    g1, dis = _tc(_disg_body, (N_PAD, D), (N_PAD, 1))(h1, degp)
    p1 = _sc_agg(g1, ei_a)
    (g2,) = _tc(_mid_body, (N_PAD, D))(p1, g1, dis, b1r, W2)
    p2 = _sc_agg(g2, ei_a)
    (out,) = _tc(_fin_body, (N_PAD, D))(p2, g2, dis, b2r)
    return out[:N_NODES]


# speedup vs baseline: 23.9227x; 2.5695x over previous
import functools

import jax
import jax.numpy as jnp
from jax import lax
from jax.experimental import pallas as pl
from jax.experimental.pallas import tpu as pltpu
from jax.experimental.pallas import tpu_sc as plsc

N_NODES = 10000
D = 128
N_EDGES = 320000

NC = 2
NS = 16
CHUNK = 128
CPW = 80
HALF = CPW // 2
E_PAD = NC * NS * CPW * CHUNK
N_PAD = 10240
ROWS_PS = N_PAD // NS

_MESH = dict(core_axis_name="c", subcore_axis_name="s", num_cores=NC,
             num_subcores=NS)


def _deg_body(dst_hbm, out_hbm, dst_v, ones_v, zbuf, acc_sh):
    ci = lax.axis_index("c")
    si = lax.axis_index("s")
    pltpu.sync_copy(dst_hbm.at[ci, si], dst_v)
    @pl.loop(0, CHUNK)
    def _(r):
        @pl.loop(0, D, step=16)
        def _(j):
            zbuf[r, pl.ds(j, 16)] = jnp.zeros((16,), jnp.float32)
            ones_v[r, pl.ds(j, 16)] = jnp.ones((16,), jnp.float32)
    for z in range(ROWS_PS // CHUNK):
        pltpu.sync_copy(zbuf, acc_sh.at[pl.ds(si * ROWS_PS + z * CHUNK, CHUNK)])
    plsc.subcore_barrier()
    @pl.loop(0, CPW)
    def _(c):
        pltpu.sync_copy(ones_v, acc_sh.at[dst_v.at[c]], add=True)
    plsc.subcore_barrier()
    pltpu.sync_copy(acc_sh.at[pl.ds(si * ROWS_PS, ROWS_PS)],
                    out_hbm.at[ci].at[pl.ds(si * ROWS_PS, ROWS_PS)])


NBUF = 2


def _agg_body(g_hbm, ei_hbm, out_hbm, idx_v, rows, acc_sh, *sems):
    gsem = sems[:NBUF]
    ssem = sems[NBUF:]
    ci = lax.axis_index("c")
    si = lax.axis_index("s")
    @pl.loop(0, CHUNK)
    def _(r):
        @pl.loop(0, D, step=16)
        def _(j):
            rows[0, r, pl.ds(j, 16)] = jnp.zeros((16,), jnp.float32)
    for z in range(ROWS_PS // CHUNK):
        pltpu.sync_copy(rows.at[0],
                        acc_sh.at[pl.ds(si * ROWS_PS + z * CHUNK, CHUNK)])
    plsc.subcore_barrier()
    for h in range(CPW // HALF):
        pltpu.sync_copy(ei_hbm.at[ci, si, pl.ds(h * HALF, HALF)], idx_v)
        for b in range(NBUF):
            pltpu.async_copy(g_hbm.at[idx_v.at[b, 0]], rows.at[b], gsem[b])

        @pl.loop(0, HALF, step=NBUF)
        def _(c0):
            for b in range(NBUF):
                pltpu.make_async_copy(g_hbm.at[idx_v.at[c0 + b, 0]],
                                      rows.at[b], gsem[b]).wait()
                pltpu.async_copy(rows.at[b], acc_sh.at[idx_v.at[c0 + b, 1]],
                                 ssem[b], add=True)
            for b in range(NBUF):
                c = c0 + b + NBUF
                @pl.when(c < HALF)
                def _():
                    pltpu.make_async_copy(rows.at[b],
                                          acc_sh.at[idx_v.at[c0 + b, 1]],
                                          ssem[b]).wait()
                    pltpu.async_copy(g_hbm.at[idx_v.at[c, 0]], rows.at[b],
                                     gsem[b])

        for b in range(NBUF):
            pltpu.make_async_copy(rows.at[b],
                                  acc_sh.at[idx_v.at[HALF - NBUF + b, 1]],
                                  ssem[b]).wait()
    plsc.subcore_barrier()
    pltpu.sync_copy(acc_sh.at[pl.ds(si * ROWS_PS, ROWS_PS)],
                    out_hbm.at[ci].at[pl.ds(si * ROWS_PS, ROWS_PS)])


def _sc_deg(dst_a):
    return pl.kernel(
        _deg_body,
        out_type=jax.ShapeDtypeStruct((NC, N_PAD, D), jnp.float32),
        mesh=plsc.VectorSubcoreMesh(**_MESH),
        scratch_types=[
            pltpu.VMEM((CPW, CHUNK), jnp.int32),
            pltpu.VMEM((CHUNK, D), jnp.float32),
            pltpu.VMEM((CHUNK, D), jnp.float32),
            pltpu.VMEM_SHARED((N_PAD, D), jnp.float32),
        ],
    )(dst_a)


def _sc_agg(g, ei_a):
    return pl.kernel(
        _agg_body,
        out_type=jax.ShapeDtypeStruct((NC, N_PAD, D), jnp.float32),
        mesh=plsc.VectorSubcoreMesh(**_MESH),
        scratch_types=[
            pltpu.VMEM((HALF, 2, CHUNK), jnp.int32),
            pltpu.VMEM((NBUF, CHUNK, D), jnp.float32),
            pltpu.VMEM_SHARED((N_PAD, D), jnp.float32),
        ] + [pltpu.SemaphoreType.DMA] * (2 * NBUF),
    )(g, ei_a)


def _mm_body(x_ref, w_ref, o_ref):
    o_ref[...] = jnp.dot(x_ref[...], w_ref[...],
                         preferred_element_type=jnp.float32)


def _disg_body(h_ref, dp_ref, g_ref, dis_ref):
    deg = dp_ref[0, :, 0:1] + dp_ref[1, :, 0:1] + 1.0
    dis = lax.rsqrt(deg)
    dis_ref[...] = dis
    g_ref[...] = h_ref[...] * dis


def _mid_body(p_ref, g1_ref, dis_ref, b1_ref, w2_ref, g2_ref):
    agg = p_ref[0] + p_ref[1] + g1_ref[...]
    h = jnp.maximum(agg * dis_ref[...] + b1_ref[...], 0.0)
    h2 = jnp.dot(h, w2_ref[...], preferred_element_type=jnp.float32)
    g2_ref[...] = h2 * dis_ref[...]


def _fin_body(p_ref, g2_ref, dis_ref, b2_ref, o_ref):
    o_ref[...] = ((p_ref[0] + p_ref[1] + g2_ref[...]) * dis_ref[...]
                  + b2_ref[...])


def _tc(body, *out_shapes):
    return pl.pallas_call(
        body,
        out_shape=[jax.ShapeDtypeStruct(s, jnp.float32) for s in out_shapes],
    )


@jax.jit
def kernel(x, edge_index, W1, b1, W2, b2):
    ei = edge_index.astype(jnp.int32)
    r = jnp.arange(E_PAD - N_EDGES, dtype=jnp.int32)
    pad = jnp.stack([r % N_NODES, N_NODES + r % (N_PAD - N_NODES)])
    ei_a = (jnp.concatenate([ei, pad], axis=1)
            .reshape(2, NC, NS, CPW, CHUNK).transpose(1, 2, 3, 0, 4))
    x_pad = jnp.pad(x, ((0, N_PAD - N_NODES), (0, 0)))
    b1r = b1.reshape(1, D)
    b2r = b2.reshape(1, D)

    degp = _sc_deg(ei_a[:, :, :, 1])
    (h1,) = _tc(_mm_body, (N_PAD, D))(x_pad, W1)
    g1, dis = _tc(_disg_body, (N_PAD, D), (N_PAD, 1))(h1, degp)
    p1 = _sc_agg(g1, ei_a)
    (g2,) = _tc(_mid_body, (N_PAD, D))(p1, g1, dis, b1r, W2)
    p2 = _sc_agg(g2, ei_a)
    (out,) = _tc(_fin_body, (N_PAD, D))(p2, g2, dis, b2r)
    return out[:N_NODES]


# speedup vs baseline: 27.0119x; 1.1291x over previous
import functools

import jax
import jax.numpy as jnp
from jax import lax
from jax.experimental import pallas as pl
from jax.experimental.pallas import tpu as pltpu
from jax.experimental.pallas import tpu_sc as plsc

N_NODES = 10000
D = 128
N_EDGES = 320000

NC = 2
NS = 16
EPS = 10240
E_PAD = NC * NS * EPS
N_PAD = 10240
ROWS_PS = N_PAD // NS

CHUNK = 128
CPW = EPS // CHUNK

CHUNK_A = 64
NCH = EPS // CHUNK_A
NLOAD = 4
QTR = NCH // NLOAD
NBUF = 4

_MESH = dict(core_axis_name="c", subcore_axis_name="s", num_cores=NC,
             num_subcores=NS)


def _deg_body(dst_hbm, out_hbm, dst_v, ones_v, zbuf, acc_sh):
    ci = lax.axis_index("c")
    si = lax.axis_index("s")
    pltpu.sync_copy(dst_hbm.at[ci, si], dst_v)
    @pl.loop(0, CHUNK)
    def _(r):
        @pl.loop(0, D, step=16)
        def _(j):
            zbuf[r, pl.ds(j, 16)] = jnp.zeros((16,), jnp.float32)
            ones_v[r, pl.ds(j, 16)] = jnp.ones((16,), jnp.float32)
    for z in range(ROWS_PS // CHUNK):
        pltpu.sync_copy(zbuf, acc_sh.at[pl.ds(si * ROWS_PS + z * CHUNK, CHUNK)])
    plsc.subcore_barrier()
    @pl.loop(0, CPW)
    def _(c):
        pltpu.sync_copy(ones_v, acc_sh.at[dst_v.at[c]], add=True)
    plsc.subcore_barrier()
    pltpu.sync_copy(acc_sh.at[pl.ds(si * ROWS_PS, ROWS_PS)],
                    out_hbm.at[ci].at[pl.ds(si * ROWS_PS, ROWS_PS)])


def _agg_body(g_hbm, ei_hbm, out_hbm, idx_v, rows, acc_sh, *sems):
    gsem = sems[:NBUF]
    ssem = sems[NBUF:]
    ci = lax.axis_index("c")
    si = lax.axis_index("s")
    @pl.loop(0, CHUNK_A)
    def _(r):
        @pl.loop(0, D, step=16)
        def _(j):
            rows[0, r, pl.ds(j, 16)] = jnp.zeros((16,), jnp.float32)
    for z in range(ROWS_PS // CHUNK_A):
        pltpu.sync_copy(rows.at[0],
                        acc_sh.at[pl.ds(si * ROWS_PS + z * CHUNK_A, CHUNK_A)])
    plsc.subcore_barrier()
    for h in range(NLOAD):
        pltpu.sync_copy(ei_hbm.at[0, ci, si, pl.ds(h * QTR, QTR)],
                        idx_v.at[0])
        pltpu.sync_copy(ei_hbm.at[1, ci, si, pl.ds(h * QTR, QTR)],
                        idx_v.at[1])
        for b in range(NBUF):
            pltpu.async_copy(g_hbm.at[idx_v.at[0, b]], rows.at[b], gsem[b])

        @pl.loop(0, QTR, step=NBUF)
        def _(c0):
            for b in range(NBUF):
                pltpu.make_async_copy(g_hbm.at[idx_v.at[0, c0 + b]],
                                      rows.at[b], gsem[b]).wait()
                pltpu.async_copy(rows.at[b], acc_sh.at[idx_v.at[1, c0 + b]],
                                 ssem[b], add=True)
            for b in range(NBUF):
                c = c0 + b + NBUF
                @pl.when(c < QTR)
                def _():
                    pltpu.make_async_copy(rows.at[b],
                                          acc_sh.at[idx_v.at[1, c0 + b]],
                                          ssem[b]).wait()
                    pltpu.async_copy(g_hbm.at[idx_v.at[0, c]], rows.at[b],
                                     gsem[b])

        for b in range(NBUF):
            pltpu.make_async_copy(rows.at[b],
                                  acc_sh.at[idx_v.at[1, QTR - NBUF + b]],
                                  ssem[b]).wait()
    plsc.subcore_barrier()
    pltpu.sync_copy(acc_sh.at[pl.ds(si * ROWS_PS, ROWS_PS)],
                    out_hbm.at[ci].at[pl.ds(si * ROWS_PS, ROWS_PS)])


def _sc_deg(dst_a):
    return pl.kernel(
        _deg_body,
        out_type=jax.ShapeDtypeStruct((NC, N_PAD, D), jnp.float32),
        mesh=plsc.VectorSubcoreMesh(**_MESH),
        scratch_types=[
            pltpu.VMEM((CPW, CHUNK), jnp.int32),
            pltpu.VMEM((CHUNK, D), jnp.float32),
            pltpu.VMEM((CHUNK, D), jnp.float32),
            pltpu.VMEM_SHARED((N_PAD, D), jnp.float32),
        ],
    )(dst_a)


def _sc_agg(g, ei_a):
    return pl.kernel(
        _agg_body,
        out_type=jax.ShapeDtypeStruct((NC, N_PAD, D), jnp.float32),
        mesh=plsc.VectorSubcoreMesh(**_MESH),
        scratch_types=[
            pltpu.VMEM((2, QTR, CHUNK_A), jnp.int32),
            pltpu.VMEM((NBUF, CHUNK_A, D), jnp.float32),
            pltpu.VMEM_SHARED((N_PAD, D), jnp.float32),
        ] + [pltpu.SemaphoreType.DMA] * (2 * NBUF),
    )(g, ei_a)


def _mm_body(x_ref, w_ref, o_ref):
    o_ref[...] = jnp.dot(x_ref[...], w_ref[...],
                         preferred_element_type=jnp.float32)


def _disg_body(h_ref, dp_ref, g_ref, dis_ref):
    deg = dp_ref[0, :, 0:1] + dp_ref[1, :, 0:1] + 1.0
    dis = lax.rsqrt(deg)
    dis_ref[...] = dis
    g_ref[...] = h_ref[...] * dis


def _mid_body(p_ref, g1_ref, dis_ref, b1_ref, w2_ref, g2_ref):
    agg = p_ref[0] + p_ref[1] + g1_ref[...]
    h = jnp.maximum(agg * dis_ref[...] + b1_ref[...], 0.0)
    h2 = jnp.dot(h, w2_ref[...], preferred_element_type=jnp.float32)
    g2_ref[...] = h2 * dis_ref[...]


def _fin_body(p_ref, g2_ref, dis_ref, b2_ref, o_ref):
    o_ref[...] = ((p_ref[0] + p_ref[1] + g2_ref[...]) * dis_ref[...]
                  + b2_ref[...])


def _tc(body, *out_shapes):
    return pl.pallas_call(
        body,
        out_shape=[jax.ShapeDtypeStruct(s, jnp.float32) for s in out_shapes],
    )


@jax.jit
def kernel(x, edge_index, W1, b1, W2, b2):
    ei = edge_index.astype(jnp.int32)
    r = jnp.arange(E_PAD - N_EDGES, dtype=jnp.int32)
    pad = jnp.stack([r % N_NODES, N_NODES + r % (N_PAD - N_NODES)])
    ei2 = jnp.concatenate([ei, pad], axis=1).reshape(2, NC, NS, EPS)
    ei_a = ei2.reshape(2, NC, NS, NCH, CHUNK_A)
    dst_d = ei2[1].reshape(NC, NS, CPW, CHUNK)
    x_pad = jnp.pad(x, ((0, N_PAD - N_NODES), (0, 0)))
    b1r = b1.reshape(1, D)
    b2r = b2.reshape(1, D)

    degp = _sc_deg(dst_d)
    (h1,) = _tc(_mm_body, (N_PAD, D))(x_pad, W1)
    g1, dis = _tc(_disg_body, (N_PAD, D), (N_PAD, 1))(h1, degp)
    p1 = _sc_agg(g1, ei_a)
    (g2,) = _tc(_mid_body, (N_PAD, D))(p1, g1, dis, b1r, W2)
    p2 = _sc_agg(g2, ei_a)
    (out,) = _tc(_fin_body, (N_PAD, D))(p2, g2, dis, b2r)
    return out[:N_NODES]


# speedup vs baseline: 30.6704x; 1.1354x over previous
import functools

import jax
import jax.numpy as jnp
from jax import lax
from jax.experimental import pallas as pl
from jax.experimental.pallas import tpu as pltpu
from jax.experimental.pallas import tpu_sc as plsc

N_NODES = 10000
D = 128
N_EDGES = 320000

NC = 2
NS = 16
EPS = 10240
E_PAD = NC * NS * EPS
N_PAD = 10240
ROWS_PS = N_PAD // NS

CHUNK = 128
CPW = EPS // CHUNK

CHUNK_A = 64
NCH = EPS // CHUNK_A
NLOAD = 4
QTR = NCH // NLOAD
NBUF = 4

_MESH = dict(core_axis_name="c", subcore_axis_name="s", num_cores=NC,
             num_subcores=NS)


def _deg_body(dst_hbm, out_hbm, dst_v, hist, tmp, accl, stage_sh):
    ci = lax.axis_index("c")
    si = lax.axis_index("s")
    @pl.loop(0, N_PAD, step=16)
    def _(i):
        hist[pl.ds(i, 16)] = jnp.zeros((16,), jnp.float32)
    for h in range(2):
        pltpu.sync_copy(dst_hbm.at[ci, si, pl.ds(h * (EPS // 2), EPS // 2)],
                        dst_v)
        @pl.loop(0, EPS // 2, step=16)
        def _(e):
            plsc.addupdate_scatter(hist, [dst_v[pl.ds(e, 16)]],
                                   jnp.ones((16,), jnp.float32))
    pltpu.sync_copy(hist, stage_sh.at[si])
    plsc.subcore_barrier()
    @pl.loop(0, ROWS_PS, step=16)
    def _(i):
        accl[pl.ds(i, 16)] = jnp.zeros((16,), jnp.float32)
    for k in range(NS):
        pltpu.sync_copy(stage_sh.at[k, pl.ds(si * ROWS_PS, ROWS_PS)], tmp)
        @pl.loop(0, ROWS_PS, step=16)
        def _(i):
            accl[pl.ds(i, 16)] = accl[pl.ds(i, 16)] + tmp[pl.ds(i, 16)]
    pltpu.sync_copy(accl, out_hbm.at[ci].at[pl.ds(si * ROWS_PS, ROWS_PS)])


def _agg_body(g_hbm, ei_hbm, out_hbm, idx_v, rows, acc_sh, *sems):
    gsem = sems[:NBUF]
    ssem = sems[NBUF:]
    ci = lax.axis_index("c")
    si = lax.axis_index("s")
    @pl.loop(0, CHUNK_A)
    def _(r):
        @pl.loop(0, D, step=16)
        def _(j):
            rows[0, r, pl.ds(j, 16)] = jnp.zeros((16,), jnp.float32)
    for z in range(ROWS_PS // CHUNK_A):
        pltpu.sync_copy(rows.at[0],
                        acc_sh.at[pl.ds(si * ROWS_PS + z * CHUNK_A, CHUNK_A)])
    plsc.subcore_barrier()
    for h in range(NLOAD):
        pltpu.sync_copy(ei_hbm.at[0, ci, si, pl.ds(h * QTR, QTR)],
                        idx_v.at[0])
        pltpu.sync_copy(ei_hbm.at[1, ci, si, pl.ds(h * QTR, QTR)],
                        idx_v.at[1])
        for b in range(NBUF):
            pltpu.async_copy(g_hbm.at[idx_v.at[0, b]], rows.at[b], gsem[b])

        @pl.loop(0, QTR, step=NBUF)
        def _(c0):
            for b in range(NBUF):
                pltpu.make_async_copy(g_hbm.at[idx_v.at[0, c0 + b]],
                                      rows.at[b], gsem[b]).wait()
                pltpu.async_copy(rows.at[b], acc_sh.at[idx_v.at[1, c0 + b]],
                                 ssem[b], add=True)
            for b in range(NBUF):
                c = c0 + b + NBUF
                @pl.when(c < QTR)
                def _():
                    pltpu.make_async_copy(rows.at[b],
                                          acc_sh.at[idx_v.at[1, c0 + b]],
                                          ssem[b]).wait()
                    pltpu.async_copy(g_hbm.at[idx_v.at[0, c]], rows.at[b],
                                     gsem[b])

        for b in range(NBUF):
            pltpu.make_async_copy(rows.at[b],
                                  acc_sh.at[idx_v.at[1, QTR - NBUF + b]],
                                  ssem[b]).wait()
    plsc.subcore_barrier()
    pltpu.sync_copy(acc_sh.at[pl.ds(si * ROWS_PS, ROWS_PS)],
                    out_hbm.at[ci].at[pl.ds(si * ROWS_PS, ROWS_PS)])


def _sc_deg(dst_d):
    import dataclasses
    cp = pltpu.CompilerParams()
    if "needs_layout_passes" in pltpu.CompilerParams.__dataclass_fields__:
        cp = dataclasses.replace(cp, needs_layout_passes=False)
    return pl.kernel(
        _deg_body,
        out_type=jax.ShapeDtypeStruct((NC, N_PAD), jnp.float32),
        mesh=plsc.VectorSubcoreMesh(**_MESH),
        compiler_params=cp,
        scratch_types=[
            pltpu.VMEM((EPS // 2,), jnp.int32),
            pltpu.VMEM((N_PAD,), jnp.float32),
            pltpu.VMEM((ROWS_PS,), jnp.float32),
            pltpu.VMEM((ROWS_PS,), jnp.float32),
            pltpu.VMEM_SHARED((NS, N_PAD), jnp.float32),
        ],
    )(dst_d)


def _sc_agg(g, ei_a):
    return pl.kernel(
        _agg_body,
        out_type=jax.ShapeDtypeStruct((NC, N_PAD, D), jnp.float32),
        mesh=plsc.VectorSubcoreMesh(**_MESH),
        scratch_types=[
            pltpu.VMEM((2, QTR, CHUNK_A), jnp.int32),
            pltpu.VMEM((NBUF, CHUNK_A, D), jnp.float32),
            pltpu.VMEM_SHARED((N_PAD, D), jnp.float32),
        ] + [pltpu.SemaphoreType.DMA] * (2 * NBUF),
    )(g, ei_a)


def _mm_body(x_ref, w_ref, o_ref):
    o_ref[...] = jnp.dot(x_ref[...], w_ref[...],
                         preferred_element_type=jnp.float32)


def _disg_body(h_ref, dp_ref, g_ref, dis_ref):
    deg = dp_ref[0] + dp_ref[1] + 1.0
    dis = lax.rsqrt(deg)
    dis_ref[...] = dis
    g_ref[...] = h_ref[...] * dis


def _mid_body(p_ref, g1_ref, dis_ref, b1_ref, w2_ref, g2_ref):
    agg = p_ref[0] + p_ref[1] + g1_ref[...]
    h = jnp.maximum(agg * dis_ref[...] + b1_ref[...], 0.0)
    h2 = jnp.dot(h, w2_ref[...], preferred_element_type=jnp.float32)
    g2_ref[...] = h2 * dis_ref[...]


def _fin_body(p_ref, g2_ref, dis_ref, b2_ref, o_ref):
    o_ref[...] = ((p_ref[0] + p_ref[1] + g2_ref[...]) * dis_ref[...]
                  + b2_ref[...])


def _tc(body, *out_shapes):
    return pl.pallas_call(
        body,
        out_shape=[jax.ShapeDtypeStruct(s, jnp.float32) for s in out_shapes],
    )


@jax.jit
def kernel(x, edge_index, W1, b1, W2, b2):
    ei = edge_index.astype(jnp.int32)
    r = jnp.arange(E_PAD - N_EDGES, dtype=jnp.int32)
    pad = jnp.stack([r % N_NODES, N_NODES + r % (N_PAD - N_NODES)])
    ei2 = jnp.concatenate([ei, pad], axis=1).reshape(2, NC, NS, EPS)
    ei_a = ei2.reshape(2, NC, NS, NCH, CHUNK_A)
    dst_d = ei2[1]
    x_pad = jnp.pad(x, ((0, N_PAD - N_NODES), (0, 0)))
    b1r = b1.reshape(1, D)
    b2r = b2.reshape(1, D)

    degp = _sc_deg(dst_d).reshape(NC, N_PAD, 1)
    (h1,) = _tc(_mm_body, (N_PAD, D))(x_pad, W1)
    g1, dis = _tc(_disg_body, (N_PAD, D), (N_PAD, 1))(h1, degp)
    p1 = _sc_agg(g1, ei_a)
    (g2,) = _tc(_mid_body, (N_PAD, D))(p1, g1, dis, b1r, W2)
    p2 = _sc_agg(g2, ei_a)
    (out,) = _tc(_fin_body, (N_PAD, D))(p2, g2, dis, b2r)
    return out[:N_NODES]


# speedup vs baseline: 31.4013x; 1.0238x over previous
import functools

import jax
import jax.numpy as jnp
from jax import lax
from jax.experimental import pallas as pl
from jax.experimental.pallas import tpu as pltpu
from jax.experimental.pallas import tpu_sc as plsc

N_NODES = 10000
D = 128
N_EDGES = 320000

NC = 2
NS = 16
EPS = 10240
E_PAD = NC * NS * EPS
N_PAD = 10240
ROWS_PS = N_PAD // NS

CHUNK = 128
CPW = EPS // CHUNK

CHUNK_A = 64
NCH = EPS // CHUNK_A
NLOAD = 4
QTR = NCH // NLOAD
NBUF = 4

_MESH = dict(core_axis_name="c", subcore_axis_name="s", num_cores=NC,
             num_subcores=NS)


def _deg_body(dst_hbm, out_hbm, dst_v, hist, tmp, accl, stage_sh):
    ci = lax.axis_index("c")
    si = lax.axis_index("s")
    @pl.loop(0, N_PAD, step=16)
    def _(i):
        hist[pl.ds(i, 16)] = jnp.zeros((16,), jnp.float32)
    for h in range(2):
        pltpu.sync_copy(dst_hbm.at[ci, si, pl.ds(h * (EPS // 2), EPS // 2)],
                        dst_v)
        @pl.loop(0, EPS // 2, step=16)
        def _(e):
            plsc.addupdate_scatter(hist, [dst_v[pl.ds(e, 16)]],
                                   jnp.ones((16,), jnp.float32))
    pltpu.sync_copy(hist, stage_sh.at[si])
    plsc.subcore_barrier()
    @pl.loop(0, ROWS_PS, step=16)
    def _(i):
        accl[pl.ds(i, 16)] = jnp.zeros((16,), jnp.float32)
    for k in range(NS):
        pltpu.sync_copy(stage_sh.at[k, pl.ds(si * ROWS_PS, ROWS_PS)], tmp)
        @pl.loop(0, ROWS_PS, step=16)
        def _(i):
            accl[pl.ds(i, 16)] = accl[pl.ds(i, 16)] + tmp[pl.ds(i, 16)]
    pltpu.sync_copy(accl, out_hbm.at[ci].at[pl.ds(si * ROWS_PS, ROWS_PS)])


def _agg_body(g_hbm, ei_hbm, out_hbm, idx_v, rows, acc_sh, *sems):
    gsem = sems[:NBUF]
    ssem = sems[NBUF:]
    ci = lax.axis_index("c")
    si = lax.axis_index("s")
    pltpu.sync_copy(ei_hbm.at[0, ci, si, pl.ds(0, QTR)], idx_v.at[0])
    pltpu.sync_copy(ei_hbm.at[1, ci, si, pl.ds(0, QTR)], idx_v.at[1])
    for b in range(1, NBUF):
        pltpu.async_copy(g_hbm.at[idx_v.at[0, b]], rows.at[b], gsem[b])
    @pl.loop(0, CHUNK_A)
    def _(r):
        @pl.loop(0, D, step=16)
        def _(j):
            rows[0, r, pl.ds(j, 16)] = jnp.zeros((16,), jnp.float32)
    for z in range(ROWS_PS // CHUNK_A):
        pltpu.sync_copy(rows.at[0],
                        acc_sh.at[pl.ds(si * ROWS_PS + z * CHUNK_A, CHUNK_A)])
    pltpu.async_copy(g_hbm.at[idx_v.at[0, 0]], rows.at[0], gsem[0])
    plsc.subcore_barrier()
    for h in range(NLOAD):
        if h:
            pltpu.sync_copy(ei_hbm.at[0, ci, si, pl.ds(h * QTR, QTR)],
                            idx_v.at[0])
            pltpu.sync_copy(ei_hbm.at[1, ci, si, pl.ds(h * QTR, QTR)],
                            idx_v.at[1])
            for b in range(NBUF):
                pltpu.async_copy(g_hbm.at[idx_v.at[0, b]], rows.at[b],
                                 gsem[b])

        @pl.loop(0, QTR, step=NBUF)
        def _(c0):
            for b in range(NBUF):
                pltpu.make_async_copy(g_hbm.at[idx_v.at[0, c0 + b]],
                                      rows.at[b], gsem[b]).wait()
                pltpu.async_copy(rows.at[b], acc_sh.at[idx_v.at[1, c0 + b]],
                                 ssem[b], add=True)
            for b in range(NBUF):
                c = c0 + b + NBUF
                @pl.when(c < QTR)
                def _():
                    pltpu.make_async_copy(rows.at[b],
                                          acc_sh.at[idx_v.at[1, c0 + b]],
                                          ssem[b]).wait()
                    pltpu.async_copy(g_hbm.at[idx_v.at[0, c]], rows.at[b],
                                     gsem[b])

        for b in range(NBUF):
            pltpu.make_async_copy(rows.at[b],
                                  acc_sh.at[idx_v.at[1, QTR - NBUF + b]],
                                  ssem[b]).wait()
    plsc.subcore_barrier()
    pltpu.sync_copy(acc_sh.at[pl.ds(si * ROWS_PS, ROWS_PS)],
                    out_hbm.at[ci].at[pl.ds(si * ROWS_PS, ROWS_PS)])


def _sc_deg(dst_d):
    import dataclasses
    cp = pltpu.CompilerParams()
    if "needs_layout_passes" in pltpu.CompilerParams.__dataclass_fields__:
        cp = dataclasses.replace(cp, needs_layout_passes=False)
    return pl.kernel(
        _deg_body,
        out_type=jax.ShapeDtypeStruct((NC, N_PAD), jnp.float32),
        mesh=plsc.VectorSubcoreMesh(**_MESH),
        compiler_params=cp,
        scratch_types=[
            pltpu.VMEM((EPS // 2,), jnp.int32),
            pltpu.VMEM((N_PAD,), jnp.float32),
            pltpu.VMEM((ROWS_PS,), jnp.float32),
            pltpu.VMEM((ROWS_PS,), jnp.float32),
            pltpu.VMEM_SHARED((NS, N_PAD), jnp.float32),
        ],
    )(dst_d)


def _sc_agg(g, ei_a):
    return pl.kernel(
        _agg_body,
        out_type=jax.ShapeDtypeStruct((NC, N_PAD, D), jnp.float32),
        mesh=plsc.VectorSubcoreMesh(**_MESH),
        scratch_types=[
            pltpu.VMEM((2, QTR, CHUNK_A), jnp.int32),
            pltpu.VMEM((NBUF, CHUNK_A, D), jnp.float32),
            pltpu.VMEM_SHARED((N_PAD, D), jnp.float32),
        ] + [pltpu.SemaphoreType.DMA] * (2 * NBUF),
    )(g, ei_a)


def _disg_body(x_ref, w_ref, dp_ref, g_ref, dis_ref):
    h = jnp.dot(x_ref[...], w_ref[...], preferred_element_type=jnp.float32)
    deg = dp_ref[0] + dp_ref[1] + 1.0
    dis = lax.rsqrt(deg)
    dis_ref[...] = dis
    g_ref[...] = h * dis


def _mid_body(p_ref, g1_ref, dis_ref, b1_ref, w2_ref, g2_ref):
    agg = p_ref[0] + p_ref[1] + g1_ref[...]
    h = jnp.maximum(agg * dis_ref[...] + b1_ref[...], 0.0)
    h2 = jnp.dot(h, w2_ref[...], preferred_element_type=jnp.float32)
    g2_ref[...] = h2 * dis_ref[...]


def _fin_body(p_ref, g2_ref, dis_ref, b2_ref, o_ref):
    val = ((p_ref[0] + p_ref[1] + g2_ref[...]) * dis_ref[...]
           + b2_ref[...])
    o_ref[...] = val[:N_NODES]


def _tc(body, *out_shapes):
    return pl.pallas_call(
        body,
        out_shape=[jax.ShapeDtypeStruct(s, jnp.float32) for s in out_shapes],
    )


@jax.jit
def kernel(x, edge_index, W1, b1, W2, b2):
    ei = edge_index.astype(jnp.int32)
    r = jnp.arange(E_PAD - N_EDGES, dtype=jnp.int32)
    pad = jnp.stack([r % N_NODES, N_NODES + r % (N_PAD - N_NODES)])
    ei2 = jnp.concatenate([ei, pad], axis=1).reshape(2, NC, NS, EPS)
    ei_a = ei2.reshape(2, NC, NS, NCH, CHUNK_A)
    dst_d = ei2[1]
    x_pad = jnp.pad(x, ((0, N_PAD - N_NODES), (0, 0)))
    b1r = b1.reshape(1, D)
    b2r = b2.reshape(1, D)

    degp = _sc_deg(dst_d).reshape(NC, N_PAD, 1)
    g1, dis = _tc(_disg_body, (N_PAD, D), (N_PAD, 1))(x_pad, W1, degp)
    p1 = _sc_agg(g1, ei_a)
    (g2,) = _tc(_mid_body, (N_PAD, D))(p1, g1, dis, b1r, W2)
    p2 = _sc_agg(g2, ei_a)
    (out,) = _tc(_fin_body, (N_NODES, D))(p2, g2, dis, b2r)
    return out
